# baseline (speedup 1.0000x reference)
import math

import jax
import jax.numpy as jnp
from jax import lax
from jax.experimental import pallas as pl
from jax.experimental.pallas import tpu as pltpu
from jax.experimental.pallas import tpu_sc as plsc

_NUM_USERS = 100000
_POOL_SIZE = 200
_NUM_NEG = 20
_BATCH = 16384

_NC = 2
_NS = 16
_L = 16
_NW = _NC * _NS
_TOT = _BATCH * _NUM_NEG
_EPW = _TOT // _NW
_CHUNK = 128
_NCH = _EPW // _CHUNK
_VPC = _CHUNK // _L


def _tec_body(user_hbm, pool_hbm, idxk_hbm, out_hbm,
              user_v, idx_v, flat_v, out_v, sem):
    wid = lax.axis_index("s") * _NC + lax.axis_index("c")
    ebase = wid * _EPW

    pltpu.sync_copy(user_hbm, user_v)
    pltpu.sync_copy(idxk_hbm.at[pl.ds(ebase, _EPW)], idx_v)

    iota = lax.iota(jnp.int32, _L)

    def chunk_body(c, carry):
        for e in range(_VPC):
            o = c * _CHUNK + e * _L
            b_loc = (ebase + o + iota) & (_BATCH - 1)
            users = plsc.load_gather(user_v, [b_loc])
            col = idx_v[pl.ds(o, _L)]
            main = col * _MAIN + users
            tail = (_MAIN_TOT - _MAIN) + col * _TAILW + users
            flat_v[pl.ds(o, _L)] = jnp.where(users < _MAIN, main, tail)

        pltpu.async_copy(
            pool_hbm.at[flat_v.at[pl.ds(c * _CHUNK, _CHUNK)]],
            out_v.at[pl.ds(c * _CHUNK, _CHUNK)],
            sem,
        )
        return carry

    lax.fori_loop(0, _NCH, chunk_body, 0)

    pltpu.make_async_copy(pool_hbm.at[pl.ds(0, _EPW)], out_v, sem).wait()

    pltpu.sync_copy(out_v, out_hbm.at[pl.ds(ebase, _EPW)])


_MAIN = 99968
_TAILW = _NUM_USERS - _MAIN
_MAIN_TOT = _POOL_SIZE * _MAIN


def _detile_body(src_ref, tail_ref, out_ref, sem):
    copies = [
        pltpu.make_async_copy(
            src_ref.at[c, pl.ds(0, _MAIN)],
            out_ref.at[pl.ds(c * _MAIN, _MAIN)],
            sem,
        )
        for c in range(_POOL_SIZE)
    ]
    copies.append(pltpu.make_async_copy(
        tail_ref, out_ref.at[pl.ds(_MAIN_TOT, _POOL_SIZE * _TAILW)], sem))
    for cp in copies:
        cp.start()
    for cp in copies:
        cp.wait()


def _detile(pool_t, tail_flat):
    return pl.pallas_call(
        _detile_body,
        out_shape=jax.ShapeDtypeStruct((_POOL_SIZE * _NUM_USERS,), jnp.int32),
        in_specs=[pl.BlockSpec(memory_space=pl.ANY),
                  pl.BlockSpec(memory_space=pl.ANY)],
        out_specs=pl.BlockSpec(memory_space=pl.ANY),
        scratch_shapes=[pltpu.SemaphoreType.DMA],
    )(pool_t, tail_flat)


def kernel(user_id, pool, idx_k):
    mesh = plsc.VectorSubcoreMesh(core_axis_name="c", subcore_axis_name="s")
    kfn = pl.kernel(
        _tec_body,
        mesh=mesh,
        compiler_params=pltpu.CompilerParams(
            use_tc_tiling_on_sc=False, needs_layout_passes=False),
        out_type=jax.ShapeDtypeStruct((_TOT,), jnp.int32),
        scratch_types=[
            pltpu.VMEM((_BATCH,), jnp.int32),
            pltpu.VMEM((_EPW,), jnp.int32),
            pltpu.VMEM((_EPW,), jnp.int32),
            pltpu.VMEM((_EPW,), jnp.int32),
            pltpu.SemaphoreType.DMA,
        ],
    )
    pool_t = pool.T
    pool_lin_t = _detile(pool_t, pool_t[:, _MAIN:].reshape(-1))
    idxk_lin_t = idx_k.T.reshape(-1)
    neg_flat_t = kfn(user_id, pool_lin_t, idxk_lin_t)
    neg_items = neg_flat_t.reshape(_NUM_NEG, _BATCH).T
    log_q = jnp.full((_BATCH, _NUM_NEG), -math.log(float(_POOL_SIZE)),
                     dtype=jnp.float32)
    return neg_items, log_q


# speedup vs baseline: 16.0467x; 16.0467x over previous
import math

import jax
import jax.numpy as jnp
from jax import lax
from jax.experimental import pallas as pl
from jax.experimental.pallas import tpu as pltpu
from jax.experimental.pallas import tpu_sc as plsc

_NUM_USERS = 100000
_POOL_SIZE = 200
_NUM_NEG = 20
_BATCH = 16384

_NC = 2
_NS = 16
_L = 16
_NW = _NC * _NS
_TOT = _BATCH * _NUM_NEG
_EPW = _TOT // _NW
_CHUNK = 128
_NCH = _EPW // _CHUNK
_VPC = _CHUNK // _L

_SC_PARAMS = pltpu.CompilerParams(
    use_tc_tiling_on_sc=False, needs_layout_passes=False)


def _idx_body(user_hbm, idxk_hbm, flat_hbm, user_v, idx_v, flat_v):
    wid = lax.axis_index("s") * _NC + lax.axis_index("c")
    ebase = wid * _EPW

    pltpu.sync_copy(user_hbm, user_v)
    pltpu.sync_copy(idxk_hbm.at[pl.ds(ebase, _EPW)], idx_v)

    iota = lax.iota(jnp.int32, _L)

    def chunk_body(c, carry):
        for e in range(_VPC):
            o = c * _CHUNK + e * _L
            b_loc = (ebase + o + iota) & (_BATCH - 1)
            users = plsc.load_gather(user_v, [b_loc])
            col = idx_v[pl.ds(o, _L)]
            flat_v[pl.ds(o, _L)] = col * _NUM_USERS + users
        return carry

    lax.fori_loop(0, _NCH, chunk_body, 0)

    pltpu.sync_copy(flat_v, flat_hbm.at[pl.ds(ebase, _EPW)])


def _gather_body(pool_hbm, flat_hbm, out_hbm, flat_v, out_v, sem):
    wid = lax.axis_index("s") * _NC + lax.axis_index("c")
    ebase = wid * _EPW

    pltpu.sync_copy(flat_hbm.at[pl.ds(ebase, _EPW)], flat_v)

    def chunk_body(c, carry):
        pltpu.async_copy(
            pool_hbm.at[flat_v.at[pl.ds(c * _CHUNK, _CHUNK)]],
            out_v.at[pl.ds(c * _CHUNK, _CHUNK)],
            sem,
        )
        return carry

    lax.fori_loop(0, _NCH, chunk_body, 0)

    pltpu.make_async_copy(pool_hbm.at[pl.ds(0, _EPW)], out_v, sem).wait()

    pltpu.sync_copy(out_v, out_hbm.at[pl.ds(ebase, _EPW)])


def kernel(user_id, pool, idx_k):
    mesh = plsc.VectorSubcoreMesh(core_axis_name="c", subcore_axis_name="s")
    idx_kfn = pl.kernel(
        _idx_body,
        mesh=mesh,
        compiler_params=_SC_PARAMS,
        out_type=jax.ShapeDtypeStruct((_TOT,), jnp.int32),
        scratch_types=[
            pltpu.VMEM((_BATCH,), jnp.int32),
            pltpu.VMEM((_EPW,), jnp.int32),
            pltpu.VMEM((_EPW,), jnp.int32),
        ],
    )
    gather_kfn = pl.kernel(
        _gather_body,
        mesh=mesh,
        compiler_params=_SC_PARAMS,
        out_type=jax.ShapeDtypeStruct((_TOT,), jnp.int32),
        scratch_types=[
            pltpu.VMEM((_EPW,), jnp.int32),
            pltpu.VMEM((_EPW,), jnp.int32),
            pltpu.SemaphoreType.DMA,
        ],
    )
    pool_lin_t = pool.T.reshape(-1)
    idxk_lin_t = idx_k.T.reshape(-1)
    flat_idx = idx_kfn(user_id, idxk_lin_t)
    neg_flat_t = gather_kfn(pool_lin_t, flat_idx)
    neg_items = neg_flat_t.reshape(_NUM_NEG, _BATCH).T
    log_q = jnp.full((_BATCH, _NUM_NEG), -math.log(float(_POOL_SIZE)),
                     dtype=jnp.float32)
    return neg_items, log_q
